# baseline (speedup 1.0000x reference)
import jax
import jax.numpy as jnp
from jax.experimental import pallas as pl

N = 10000
G = 128
HEADS = 8
HID = 64


def _layer(x, src, dst, W, a_s, a_d, b, heads, out_ch):
    n = x.shape[0]
    h = (x @ W).reshape(n, heads, out_ch)
    alpha_src = (h * a_s[None, :, :]).sum(-1)
    alpha_dst = (h * a_d[None, :, :]).sum(-1)
    e = alpha_src[src] + alpha_dst[dst]
    e = jax.nn.leaky_relu(e, 0.2)
    e_max = jax.ops.segment_max(e, dst, num_segments=n)
    e_max = jnp.where(jnp.isfinite(e_max), e_max, 0.0)
    p = jnp.exp(e - e_max[dst])
    denom = jax.ops.segment_sum(p, dst, num_segments=n)
    alpha = p / (denom[dst] + 1e-16)
    msg = h[src] * alpha[:, :, None]
    out = jax.ops.segment_sum(msg, dst, num_segments=n)
    return out.reshape(n, heads * out_ch) + b


def kernel(x, edge_index, batch, W1, a_s1, a_d1, b1, W2, a_s2, a_d2, b2, W3, a_s3, a_d3, b3, linW, linb):
    n = x.shape[0]
    loops = jnp.arange(n, dtype=edge_index.dtype)
    src = jnp.concatenate([edge_index[0], loops])
    dst = jnp.concatenate([edge_index[1], loops])
    h = _layer(x, src, dst, W1, a_s1, a_d1, b1, HEADS, HID)
    h = jax.nn.elu(h)
    h = _layer(h, src, dst, W2, a_s2, a_d2, b2, HEADS, HID)
    h = jax.nn.elu(h)
    h = _layer(h, src, dst, W3, a_s3, a_d3, b3, 1, HID)
    sums = jax.ops.segment_sum(h, batch, num_segments=G)
    counts = jax.ops.segment_sum(jnp.ones((n,), h.dtype), batch, num_segments=G)
    pooled = sums / jnp.maximum(counts, 1.0)[:, None]
    return pooled @ linW + linb


# speedup vs baseline: 15.9436x; 15.9436x over previous
import functools

import jax
import jax.numpy as jnp
from jax import lax
from jax.experimental import pallas as pl
from jax.experimental.pallas import tpu as pltpu
from jax.experimental.pallas import tpu_sc as plsc

N = 10000
E = 320000
D = 128
HID = 64
HEADS = 8
G = 128

E2 = E + N
NP = 10240
EP = 331776
CH = 128
RB = 1024
NS = 16
LEAK = 0.2

f32 = jnp.float32
i32 = jnp.int32

_MESH = plsc.VectorSubcoreMesh(core_axis_name="c", subcore_axis_name="s")


def _dense_body(Bin, F, H, xb_ref, w_ref, asf_ref, adf_ref, h_ref, alph_ref):
    r = jnp.dot(xb_ref[0], w_ref[0:128, :], preferred_element_type=f32)
    for j in range(1, Bin):
        r += jnp.dot(xb_ref[j], w_ref[128 * j:128 * (j + 1), :],
                     preferred_element_type=f32)
    if F == 512:
        for j in range(4):
            h_ref[j] = r[:, 128 * j:128 * (j + 1)]
    else:
        h_ref[...] = jnp.concatenate(
            [r, jnp.zeros((RB, 128 - F), f32)], axis=1)
    rs = r * asf_ref[...]
    rd = r * adf_ref[...]
    cols = []
    for h in range(H):
        cols.append(jnp.sum(rs[:, HID * h:HID * (h + 1)], axis=1,
                            keepdims=True))
    if H < 8:
        cols.append(jnp.zeros((RB, 8 - H), f32))
    for h in range(H):
        cols.append(jnp.sum(rd[:, HID * h:HID * (h + 1)], axis=1,
                            keepdims=True))
    if H < 8:
        cols.append(jnp.zeros((RB, 8 - H), f32))
    cols.append(jnp.zeros((RB, 112), f32))
    alph_ref[...] = jnp.concatenate(cols, axis=1)


def _dense(xb, Wp, asf, adf, F, H):
    Bin = xb.shape[0]
    if F == 512:
        h_shape = jax.ShapeDtypeStruct((4, NP, 128), f32)
        h_spec = pl.BlockSpec((4, RB, 128), lambda i: (0, i, 0))
    else:
        h_shape = jax.ShapeDtypeStruct((NP, 128), f32)
        h_spec = pl.BlockSpec((RB, 128), lambda i: (i, 0))
    return pl.pallas_call(
        functools.partial(_dense_body, Bin, F, H),
        grid=(NP // RB,),
        in_specs=[
            pl.BlockSpec((Bin, RB, 128), lambda i: (0, i, 0)),
            pl.BlockSpec((Bin * 128, F), lambda i: (0, 0)),
            pl.BlockSpec((1, F), lambda i: (0, 0)),
            pl.BlockSpec((1, F), lambda i: (0, 0)),
        ],
        out_specs=[h_spec, pl.BlockSpec((RB, 128), lambda i: (i, 0))],
        out_shape=[h_shape, jax.ShapeDtypeStruct((NP, 128), f32)],
    )(xb, Wp, asf, adf)


def _edge_weights_body(src_h, dst_h, asA, adA, asB, adB, zzN,
                       w_out, den_out,
                       tas, tad, sbuf, dbuf, w0, w1, w2, w3,
                       d0, d1, d2, d3, sem):
    cid = lax.axis_index("c")
    sid = lax.axis_index("s")
    ept = EP // NS
    rows_per_tile = NP // NS
    wbufs = (w0, w1, w2, w3)
    dens = (d0, d1, d2, d3)

    @pl.when(cid == 0)
    def _():
        pltpu.sync_copy(asA, tas)
        pltpu.sync_copy(adA, tad)

    @pl.when(cid == 1)
    def _():
        pltpu.sync_copy(asB, tas)
        pltpu.sync_copy(adB, tad)

    rslice = pl.ds(sid * rows_per_tile, rows_per_tile)
    for hh in range(4):
        pltpu.sync_copy(zzN.at[rslice], dens[hh].at[rslice])
    plsc.subcore_barrier()

    def chunk(i, _):
        gbase = sid * ept + i * CH
        pltpu.sync_copy(src_h.at[pl.ds(gbase, CH)], sbuf)
        pltpu.sync_copy(dst_h.at[pl.ds(gbase, CH)], dbuf)

        def vec(j, _):
            s16 = sbuf[pl.ds(j * 16, 16)] * 4
            d16 = dbuf[pl.ds(j * 16, 16)] * 4
            ids = gbase + j * 16 + lax.iota(i32, 16)
            mf = jnp.clip(E2 - ids, 0, 1).astype(f32)
            for hh in range(4):
                e = (plsc.load_gather(tas, [s16 + hh])
                     + plsc.load_gather(tad, [d16 + hh]))
                e = jnp.maximum(e, LEAK * e)
                wbufs[hh][pl.ds(j * 16, 16)] = jnp.exp(e) * mf
            return 0

        lax.fori_loop(0, CH // 16, vec, 0)
        for hh in range(4):
            pltpu.sync_copy(wbufs[hh], w_out.at[cid, hh, pl.ds(gbase, CH)])
            pltpu.sync_copy(wbufs[hh], dens[hh].at[dbuf], add=True)
        return 0

    lax.fori_loop(0, ept // CH, chunk, 0)
    plsc.subcore_barrier()
    for hh in range(4):
        pltpu.sync_copy(dens[hh].at[rslice], den_out.at[cid, hh, rslice])


def _edge_weights(srcP, dstP, asA, adA, asB, adB, zzN):
    k = pl.kernel(
        _edge_weights_body,
        out_type=[jax.ShapeDtypeStruct((2, 4, EP), f32),
                  jax.ShapeDtypeStruct((2, 4, NP), f32)],
        mesh=_MESH,
        compiler_params=pltpu.CompilerParams(needs_layout_passes=False),
        scratch_types=[
            pltpu.VMEM((NP * 4,), f32),
            pltpu.VMEM((NP * 4,), f32),
            pltpu.VMEM((CH,), i32),
            pltpu.VMEM((CH,), i32),
            pltpu.VMEM((CH,), f32),
            pltpu.VMEM((CH,), f32),
            pltpu.VMEM((CH,), f32),
            pltpu.VMEM((CH,), f32),
            pltpu.VMEM_SHARED((NP,), f32),
            pltpu.VMEM_SHARED((NP,), f32),
            pltpu.VMEM_SHARED((NP,), f32),
            pltpu.VMEM_SHARED((NP,), f32),
            pltpu.SemaphoreType.DMA,
        ],
    )
    return k(srcP, dstP, asA, adA, asB, adB, zzN)


def _edge_weights1_body(src_h, dst_h, as1, ad1, zzN,
                        w_out, den_out,
                        tas, tad, sbuf, dbuf, wflat, den_sp, sem):
    cid = lax.axis_index("c")
    sid = lax.axis_index("s")
    ept = EP // (2 * NS)
    rows_per_tile = NP // NS

    pltpu.sync_copy(as1, tas)
    pltpu.sync_copy(ad1, tad)
    rslice = pl.ds(sid * rows_per_tile, rows_per_tile)
    pltpu.sync_copy(zzN.at[rslice], den_sp.at[rslice])
    plsc.subcore_barrier()

    tbase = (cid * NS + sid) * ept

    def chunk(i, _):
        gbase = tbase + i * CH
        pltpu.sync_copy(src_h.at[pl.ds(gbase, CH)], sbuf)
        pltpu.sync_copy(dst_h.at[pl.ds(gbase, CH)], dbuf)

        def vec(j, _):
            s16 = sbuf[pl.ds(j * 16, 16)]
            d16 = dbuf[pl.ds(j * 16, 16)]
            ids = gbase + j * 16 + lax.iota(i32, 16)
            mf = jnp.clip(E2 - ids, 0, 1).astype(f32)
            e = plsc.load_gather(tas, [s16]) + plsc.load_gather(tad, [d16])
            e = jnp.maximum(e, LEAK * e)
            wflat[pl.ds(j * 16, 16)] = jnp.exp(e) * mf
            return 0

        lax.fori_loop(0, CH // 16, vec, 0)
        pltpu.sync_copy(wflat, w_out.at[pl.ds(gbase, CH)])
        pltpu.sync_copy(wflat, den_sp.at[dbuf], add=True)
        return 0

    lax.fori_loop(0, ept // CH, chunk, 0)
    plsc.subcore_barrier()
    pltpu.sync_copy(den_sp.at[rslice], den_out.at[cid, rslice])


def _edge_weights1(srcP, dstP, as1, ad1, zzN):
    k = pl.kernel(
        _edge_weights1_body,
        out_type=[jax.ShapeDtypeStruct((EP,), f32),
                  jax.ShapeDtypeStruct((2, NP), f32)],
        mesh=_MESH,
        compiler_params=pltpu.CompilerParams(needs_layout_passes=False),
        scratch_types=[
            pltpu.VMEM((NP,), f32),
            pltpu.VMEM((NP,), f32),
            pltpu.VMEM((CH,), i32),
            pltpu.VMEM((CH,), i32),
            pltpu.VMEM((CH,), f32),
            pltpu.VMEM_SHARED((NP,), f32),
            pltpu.SemaphoreType.DMA,
        ],
    )
    return k(srcP, dstP, as1, ad1, zzN)


def _msg_body(src_h, dst_h, wA, wB, hA, hB, denA, denB,
              bA, bB, zz,
              out,
              sbuf, dbuf, wbuf, rowbuf, accb, denb, bb, acc_sp, sem):
    cid = lax.axis_index("c")
    sid = lax.axis_index("s")
    ept = EP // NS
    rows_per_tile = NP // NS
    rbase = sid * rows_per_tile

    def run(w_t, h_t, den_t, b_t, slot):
        pltpu.sync_copy(zz.at[pl.ds(rbase, rows_per_tile)],
                        acc_sp.at[pl.ds(rbase, rows_per_tile)])
        plsc.subcore_barrier()

        def chunk(i, _):
            gbase = sid * ept + i * CH
            pltpu.sync_copy(src_h.at[pl.ds(gbase, CH)], sbuf)
            pltpu.sync_copy(dst_h.at[pl.ds(gbase, CH)], dbuf)
            pltpu.sync_copy(w_t.at[pl.ds(2 * gbase, 2 * CH)], wbuf)
            pltpu.async_copy(h_t.at[sbuf], rowbuf, sem).wait()

            def grp(g, _):
                wv = wbuf[pl.ds(g * 16, 16)]
                for m in range(8):
                    j = g * 8 + m
                    w0 = jnp.full((16,), wv[2 * m], f32)
                    w1 = jnp.full((16,), wv[2 * m + 1], f32)
                    for k in range(4):
                        rowbuf[j, pl.ds(k * 16, 16)] = (
                            rowbuf[j, pl.ds(k * 16, 16)] * w0)
                    for k in range(4, 8):
                        rowbuf[j, pl.ds(k * 16, 16)] = (
                            rowbuf[j, pl.ds(k * 16, 16)] * w1)
                return 0

            lax.fori_loop(0, CH // 8, grp, 0)
            pltpu.sync_copy(rowbuf, acc_sp.at[dbuf], add=True)
            return 0

        lax.fori_loop(0, ept // CH, chunk, 0)
        plsc.subcore_barrier()

        pltpu.sync_copy(b_t, bb)

        def ep(i, _):
            rb = rbase + i * 128
            pltpu.sync_copy(acc_sp.at[pl.ds(rb, 128)], accb)
            pltpu.sync_copy(den_t.at[pl.ds(2 * rb, 256)], denb)

            def rgrp(g, _):
                dv = denb[pl.ds(g * 16, 16)]
                invv = 1.0 / (dv + 1e-16)
                for m in range(8):
                    r = g * 8 + m
                    inv0 = jnp.full((16,), invv[2 * m], f32)
                    inv1 = jnp.full((16,), invv[2 * m + 1], f32)
                    for k in range(8):
                        inv = inv0 if k < 4 else inv1
                        v = (accb[r, pl.ds(k * 16, 16)] * inv
                             + bb[pl.ds(k * 16, 16)])
                        v = (jnp.maximum(v, 0.0)
                             + jnp.minimum(
                                 jnp.exp(jnp.minimum(v, 0.0)) - 1.0, 0.0))
                        accb[r, pl.ds(k * 16, 16)] = v
                return 0

            lax.fori_loop(0, 16, rgrp, 0)
            pltpu.sync_copy(accb, out.at[slot, pl.ds(rb, 128)])
            return 0

        lax.fori_loop(0, rows_per_tile // 128, ep, 0)

    @pl.when(cid == 0)
    def _():
        run(wA, hA, denA, bA, 0)

    @pl.when(cid == 1)
    def _():
        run(wB, hB, denB, bB, 1)


def _msg(srcP, dstP, wA, wB, hA, hB, denA, denB, bA, bB, zz):
    k = pl.kernel(
        _msg_body,
        out_type=jax.ShapeDtypeStruct((2, NP, 128), f32),
        mesh=_MESH,
        scratch_types=[
            pltpu.VMEM((CH,), i32),
            pltpu.VMEM((CH,), i32),
            pltpu.VMEM((2 * CH,), f32),
            pltpu.VMEM((CH, 128), f32),
            pltpu.VMEM((128, 128), f32),
            pltpu.VMEM((256,), f32),
            pltpu.VMEM((128,), f32),
            pltpu.VMEM_SHARED((NP, 128), f32),
            pltpu.SemaphoreType.DMA,
        ],
    )
    return k(srcP, dstP, wA, wB, hA, hB, denA, denB, bA, bB, zz)


def _msg1_body(src_h, dst_h, w1, h3, zz,
               out,
               sbuf, dbuf, wbuf, rowbuf, acc_sp, sem):
    cid = lax.axis_index("c")
    sid = lax.axis_index("s")
    ept = EP // (2 * NS)
    rows_per_tile = NP // NS
    rbase = sid * rows_per_tile
    rslice = pl.ds(rbase, rows_per_tile)

    pltpu.sync_copy(zz.at[rslice], acc_sp.at[rslice])
    plsc.subcore_barrier()

    tbase = (cid * NS + sid) * ept

    def chunk(i, _):
        gbase = tbase + i * CH
        pltpu.sync_copy(src_h.at[pl.ds(gbase, CH)], sbuf)
        pltpu.sync_copy(dst_h.at[pl.ds(gbase, CH)], dbuf)
        pltpu.sync_copy(w1.at[pl.ds(gbase, CH)], wbuf)
        pltpu.async_copy(h3.at[sbuf], rowbuf, sem).wait()

        def grp(g, _):
            wv = wbuf[pl.ds(g * 16, 16)]
            for m in range(16):
                j = g * 16 + m
                w0 = jnp.full((16,), wv[m], f32)
                for k in range(4):
                    rowbuf[j, pl.ds(k * 16, 16)] = (
                        rowbuf[j, pl.ds(k * 16, 16)] * w0)
            return 0

        lax.fori_loop(0, CH // 16, grp, 0)
        pltpu.sync_copy(rowbuf, acc_sp.at[dbuf], add=True)
        return 0

    lax.fori_loop(0, ept // CH, chunk, 0)
    plsc.subcore_barrier()
    pltpu.sync_copy(acc_sp.at[rslice], out.at[cid, rslice])


def _msg1(srcP, dstP, w1, h3, zz):
    k = pl.kernel(
        _msg1_body,
        out_type=jax.ShapeDtypeStruct((2, NP, 128), f32),
        mesh=_MESH,
        scratch_types=[
            pltpu.VMEM((CH,), i32),
            pltpu.VMEM((CH,), i32),
            pltpu.VMEM((CH,), f32),
            pltpu.VMEM((CH, 128), f32),
            pltpu.VMEM_SHARED((NP, 128), f32),
            pltpu.SemaphoreType.DMA,
        ],
    )
    return k(srcP, dstP, w1, h3, zz)


def _final_body(acc_ref, den_ref, b3_ref, bb_ref, linw_ref, linb_ref,
                out_ref, ssum, scnt):
    i = pl.program_id(0)

    @pl.when(i == 0)
    def _():
        ssum[...] = jnp.zeros((G, HID), f32)
        scnt[...] = jnp.zeros((G, HID), f32)

    den = den_ref[0, :, 0:1] + den_ref[1, :, 0:1]
    out3 = (acc_ref[0] + acc_ref[1]) / (den + 1e-16) + b3_ref[...]
    oneh = (bb_ref[...] == lax.broadcasted_iota(i32, (RB, G), 1)).astype(f32)
    ssum[...] += lax.dot_general(oneh, out3, (((0,), (0,)), ((), ())),
                                 preferred_element_type=f32)
    scnt[...] += lax.dot_general(oneh, jnp.ones((RB, HID), f32),
                                 (((0,), (0,)), ((), ())),
                                 preferred_element_type=f32)

    @pl.when(i == NP // RB - 1)
    def _():
        pooled = ssum[...] / jnp.maximum(scnt[...], 1.0)
        out_ref[...] = (jnp.dot(pooled, linw_ref[...],
                                preferred_element_type=f32)
                        + linb_ref[...])


def _final(acc3p, den3, b3r, batchB, linWp, linb2):
    return pl.pallas_call(
        _final_body,
        grid=(NP // RB,),
        in_specs=[
            pl.BlockSpec((2, RB, HID), lambda i: (0, i, 0)),
            pl.BlockSpec((2, RB, 8), lambda i: (0, i, 0)),
            pl.BlockSpec((1, HID), lambda i: (0, 0)),
            pl.BlockSpec((RB, G), lambda i: (i, 0)),
            pl.BlockSpec((HID, 128), lambda i: (0, 0)),
            pl.BlockSpec((1, 128), lambda i: (0, 0)),
        ],
        out_specs=pl.BlockSpec((G, 128), lambda i: (0, 0)),
        out_shape=jax.ShapeDtypeStruct((G, 128), f32),
        scratch_shapes=[pltpu.VMEM((G, HID), f32), pltpu.VMEM((G, HID), f32)],
    )(acc3p, den3, b3r, batchB, linWp, linb2)


def _perm_rows(W):
    return jnp.concatenate(
        [W[0:128], W[256:384], W[128:256], W[384:512]], axis=0)


def kernel(x, edge_index, batch, W1, a_s1, a_d1, b1, W2, a_s2, a_d2, b2,
           W3, a_s3, a_d3, b3, linW, linb):
    loops = jnp.arange(N, dtype=edge_index.dtype)
    srcP = jnp.concatenate(
        [edge_index[0], loops,
         jnp.zeros((EP - E2,), edge_index.dtype)])
    dstP = jnp.concatenate(
        [edge_index[1], loops,
         jnp.zeros((EP - E2,), edge_index.dtype)])
    xP = jnp.pad(x, ((0, NP - N), (0, 0)))
    zz128 = jnp.zeros((NP, 128), f32)
    zzN = jnp.zeros((NP,), f32)

    def gat_layer(xb, Wp, a_s, a_d, b):
        h4, alph = _dense(xb, Wp, a_s.reshape(1, 512), a_d.reshape(1, 512),
                          512, HEADS)
        w8, den8 = _edge_weights(srcP, dstP,
                                 alph[:, 0:4].reshape(-1),
                                 alph[:, 8:12].reshape(-1),
                                 alph[:, 4:8].reshape(-1),
                                 alph[:, 12:16].reshape(-1), zzN)
        outs = []
        for q in range(2):
            outs.append(_msg(
                srcP, dstP,
                jnp.stack([w8[0, 2 * q], w8[0, 2 * q + 1]], 1).reshape(-1),
                jnp.stack([w8[1, 2 * q], w8[1, 2 * q + 1]], 1).reshape(-1),
                h4[q], h4[2 + q],
                jnp.stack([den8[0, 2 * q], den8[0, 2 * q + 1]], 1).reshape(-1),
                jnp.stack([den8[1, 2 * q], den8[1, 2 * q + 1]], 1).reshape(-1),
                b[128 * q:128 * (q + 1)], b[128 * (2 + q):128 * (3 + q)],
                zz128))
        return jnp.concatenate(outs, axis=0)

    xb1 = xP[None]
    o1 = gat_layer(xb1, W1, a_s1, a_d1, b1)
    o2 = gat_layer(o1, _perm_rows(W2), a_s2, a_d2, b2)

    h3p, alph3 = _dense(o2, jnp.pad(_perm_rows(W3), ((0, 0), (0, 0))),
                        a_s3.reshape(1, HID), a_d3.reshape(1, HID),
                        HID, 1)
    w1d, den3 = _edge_weights1(srcP, dstP, alph3[:, 0], alph3[:, 8], zzN)
    acc3p = _msg1(srcP, dstP, w1d, h3p, zz128)[:, :, :HID]
    den3b = jnp.broadcast_to(den3[:, :, None], (2, NP, 8)) + 0.0

    batchP = jnp.pad(batch, (0, NP - N), constant_values=G).astype(i32)
    batchB = jnp.broadcast_to(batchP[:, None], (NP, G))
    linWp = jnp.pad(linW, ((0, 0), (0, 128 - linW.shape[1])))
    linb2 = jnp.pad(linb.reshape(1, 1), ((0, 0), (0, 127)))
    res = _final(acc3p, den3b, b3.reshape(1, HID) + 0.0, batchB, linWp, linb2)
    return res[:, :1]


# speedup vs baseline: 22.7990x; 1.4300x over previous
import functools

import jax
import jax.numpy as jnp
from jax import lax
from jax.experimental import pallas as pl
from jax.experimental.pallas import tpu as pltpu
from jax.experimental.pallas import tpu_sc as plsc

N = 10000
E = 320000
D = 128
HID = 64
HEADS = 8
G = 128

E2 = E + N
NP = 10240
EP = 331776
CH = 128
CHM = 96
RB = 1024
NS = 16
LEAK = 0.2

f32 = jnp.float32
i32 = jnp.int32

_MESH = plsc.VectorSubcoreMesh(core_axis_name="c", subcore_axis_name="s")


def _dense_body(Bin, F, H, xb_ref, w_ref, asf_ref, adf_ref, h_ref, alph_ref):
    r = jnp.dot(xb_ref[0], w_ref[0:128, :], preferred_element_type=f32)
    for j in range(1, Bin):
        r += jnp.dot(xb_ref[j], w_ref[128 * j:128 * (j + 1), :],
                     preferred_element_type=f32)
    if F == 512:
        for j in range(4):
            h_ref[j] = r[:, 128 * j:128 * (j + 1)]
    else:
        h_ref[...] = jnp.concatenate(
            [r, jnp.zeros((RB, 128 - F), f32)], axis=1)
    rs = r * asf_ref[...]
    rd = r * adf_ref[...]
    cols = []
    for h in range(H):
        cols.append(jnp.sum(rs[:, HID * h:HID * (h + 1)], axis=1,
                            keepdims=True))
    if H < 8:
        cols.append(jnp.zeros((RB, 8 - H), f32))
    for h in range(H):
        cols.append(jnp.sum(rd[:, HID * h:HID * (h + 1)], axis=1,
                            keepdims=True))
    if H < 8:
        cols.append(jnp.zeros((RB, 8 - H), f32))
    cols.append(jnp.zeros((RB, 112), f32))
    alph_ref[...] = jnp.concatenate(cols, axis=1)


def _dense(xb, Wp, asf, adf, F, H):
    Bin = xb.shape[0]
    if F == 512:
        h_shape = jax.ShapeDtypeStruct((4, NP, 128), f32)
        h_spec = pl.BlockSpec((4, RB, 128), lambda i: (0, i, 0))
    else:
        h_shape = jax.ShapeDtypeStruct((NP, 128), f32)
        h_spec = pl.BlockSpec((RB, 128), lambda i: (i, 0))
    return pl.pallas_call(
        functools.partial(_dense_body, Bin, F, H),
        grid=(NP // RB,),
        in_specs=[
            pl.BlockSpec((Bin, RB, 128), lambda i: (0, i, 0)),
            pl.BlockSpec((Bin * 128, F), lambda i: (0, 0)),
            pl.BlockSpec((1, F), lambda i: (0, 0)),
            pl.BlockSpec((1, F), lambda i: (0, 0)),
        ],
        out_specs=[h_spec, pl.BlockSpec((RB, 128), lambda i: (i, 0))],
        out_shape=[h_shape, jax.ShapeDtypeStruct((NP, 128), f32)],
    )(xb, Wp, asf, adf)


def _edge_weights_body(src_h, dst_h, asA, adA, asB, adB, zzN,
                       w_out, den_out,
                       tas, tad, sbuf, dbuf, w0, w1, w2, w3,
                       d0, d1, d2, d3, sem):
    cid = lax.axis_index("c")
    sid = lax.axis_index("s")
    ept = EP // NS
    rows_per_tile = NP // NS
    wbufs = (w0, w1, w2, w3)
    dens = (d0, d1, d2, d3)

    @pl.when(cid == 0)
    def _():
        pltpu.sync_copy(asA, tas)
        pltpu.sync_copy(adA, tad)

    @pl.when(cid == 1)
    def _():
        pltpu.sync_copy(asB, tas)
        pltpu.sync_copy(adB, tad)

    rslice = pl.ds(sid * rows_per_tile, rows_per_tile)
    for hh in range(4):
        pltpu.sync_copy(zzN.at[rslice], dens[hh].at[rslice])
    plsc.subcore_barrier()

    def chunk(i, _):
        gbase = sid * ept + i * CH
        pltpu.sync_copy(src_h.at[pl.ds(gbase, CH)], sbuf)
        pltpu.sync_copy(dst_h.at[pl.ds(gbase, CH)], dbuf)

        def vec(j, _):
            s16 = sbuf[pl.ds(j * 16, 16)] * 4
            d16 = dbuf[pl.ds(j * 16, 16)] * 4
            ids = gbase + j * 16 + lax.iota(i32, 16)
            mf = jnp.clip(E2 - ids, 0, 1).astype(f32)
            for hh in range(4):
                e = (plsc.load_gather(tas, [s16 + hh])
                     + plsc.load_gather(tad, [d16 + hh]))
                e = jnp.maximum(e, LEAK * e)
                wbufs[hh][pl.ds(j * 16, 16)] = jnp.exp(e) * mf
            return 0

        lax.fori_loop(0, CH // 16, vec, 0)
        for hh in range(4):
            pltpu.sync_copy(wbufs[hh], w_out.at[cid, hh, pl.ds(gbase, CH)])
            pltpu.sync_copy(wbufs[hh], dens[hh].at[dbuf], add=True)
        return 0

    lax.fori_loop(0, ept // CH, chunk, 0)
    plsc.subcore_barrier()
    for hh in range(4):
        pltpu.sync_copy(dens[hh].at[rslice], den_out.at[cid, hh, rslice])


def _edge_weights(srcP, dstP, asA, adA, asB, adB, zzN):
    k = pl.kernel(
        _edge_weights_body,
        out_type=[jax.ShapeDtypeStruct((2, 4, EP), f32),
                  jax.ShapeDtypeStruct((2, 4, NP), f32)],
        mesh=_MESH,
        compiler_params=pltpu.CompilerParams(needs_layout_passes=False),
        scratch_types=[
            pltpu.VMEM((NP * 4,), f32),
            pltpu.VMEM((NP * 4,), f32),
            pltpu.VMEM((CH,), i32),
            pltpu.VMEM((CH,), i32),
            pltpu.VMEM((CH,), f32),
            pltpu.VMEM((CH,), f32),
            pltpu.VMEM((CH,), f32),
            pltpu.VMEM((CH,), f32),
            pltpu.VMEM_SHARED((NP,), f32),
            pltpu.VMEM_SHARED((NP,), f32),
            pltpu.VMEM_SHARED((NP,), f32),
            pltpu.VMEM_SHARED((NP,), f32),
            pltpu.SemaphoreType.DMA,
        ],
    )
    return k(srcP, dstP, asA, adA, asB, adB, zzN)


def _edge_weights1_body(src_h, dst_h, as1, ad1, zzN,
                        w_out, den_out,
                        tas, tad, sbuf, dbuf, wflat, den_sp, sem):
    cid = lax.axis_index("c")
    sid = lax.axis_index("s")
    ept = EP // (2 * NS)
    rows_per_tile = NP // NS

    pltpu.sync_copy(as1, tas)
    pltpu.sync_copy(ad1, tad)
    rslice = pl.ds(sid * rows_per_tile, rows_per_tile)
    pltpu.sync_copy(zzN.at[rslice], den_sp.at[rslice])
    plsc.subcore_barrier()

    tbase = (cid * NS + sid) * ept

    def chunk(i, _):
        gbase = tbase + i * CH
        pltpu.sync_copy(src_h.at[pl.ds(gbase, CH)], sbuf)
        pltpu.sync_copy(dst_h.at[pl.ds(gbase, CH)], dbuf)

        def vec(j, _):
            s16 = sbuf[pl.ds(j * 16, 16)]
            d16 = dbuf[pl.ds(j * 16, 16)]
            ids = gbase + j * 16 + lax.iota(i32, 16)
            mf = jnp.clip(E2 - ids, 0, 1).astype(f32)
            e = plsc.load_gather(tas, [s16]) + plsc.load_gather(tad, [d16])
            e = jnp.maximum(e, LEAK * e)
            wflat[pl.ds(j * 16, 16)] = jnp.exp(e) * mf
            return 0

        lax.fori_loop(0, CH // 16, vec, 0)
        pltpu.sync_copy(wflat, w_out.at[pl.ds(gbase, CH)])
        pltpu.sync_copy(wflat, den_sp.at[dbuf], add=True)
        return 0

    lax.fori_loop(0, ept // CH, chunk, 0)
    plsc.subcore_barrier()
    pltpu.sync_copy(den_sp.at[rslice], den_out.at[cid, rslice])


def _edge_weights1(srcP, dstP, as1, ad1, zzN):
    k = pl.kernel(
        _edge_weights1_body,
        out_type=[jax.ShapeDtypeStruct((EP,), f32),
                  jax.ShapeDtypeStruct((2, NP), f32)],
        mesh=_MESH,
        compiler_params=pltpu.CompilerParams(needs_layout_passes=False),
        scratch_types=[
            pltpu.VMEM((NP,), f32),
            pltpu.VMEM((NP,), f32),
            pltpu.VMEM((CH,), i32),
            pltpu.VMEM((CH,), i32),
            pltpu.VMEM((CH,), f32),
            pltpu.VMEM_SHARED((NP,), f32),
            pltpu.SemaphoreType.DMA,
        ],
    )
    return k(srcP, dstP, as1, ad1, zzN)


def _msg_body(src_h, dst_h, wA, wB, hA, hB, denA, denB,
              bA, bB, zz,
              out,
              s0, s1, s2, d0, d1, d2, w0b, w1b, w2b, r0, r1, r2,
              accb, denb, bb, acc_sp,
              sg0, sg1, sg2, ss0, ss1, ss2,
              sw0, sw1, sw2, si0, si1, si2, sj0, sj1, sj2):
    cid = lax.axis_index("c")
    sid = lax.axis_index("s")
    ept = EP // NS
    rows_per_tile = NP // NS
    rbase = sid * rows_per_tile
    sbufs, dbufs = (s0, s1, s2), (d0, d1, d2)
    wbufs, rows = (w0b, w1b, w2b), (r0, r1, r2)
    semg, sems, semw = (sg0, sg1, sg2), (ss0, ss1, ss2), (sw0, sw1, sw2)
    semsi, semdi = (si0, si1, si2), (sj0, sj1, sj2)

    def run(w_t, h_t, den_t, b_t, slot):
        pltpu.sync_copy(zz.at[pl.ds(rbase, rows_per_tile)],
                        acc_sp.at[pl.ds(rbase, rows_per_tile)])
        plsc.subcore_barrier()

        gb0 = sid * ept
        nch = ept // CHM
        nt = nch // 3

        def sidx_start(c, b):
            pltpu.async_copy(src_h.at[pl.ds(gb0 + c * CHM, CHM)],
                             sbufs[b], semsi[b])

        def sidx_wait(c, b):
            pltpu.make_async_copy(src_h.at[pl.ds(gb0 + c * CHM, CHM)],
                                  sbufs[b], semsi[b]).wait()

        def didx_start(c, b):
            pltpu.async_copy(dst_h.at[pl.ds(gb0 + c * CHM, CHM)],
                             dbufs[b], semdi[b])

        def didx_wait(c, b):
            pltpu.make_async_copy(dst_h.at[pl.ds(gb0 + c * CHM, CHM)],
                                  dbufs[b], semdi[b]).wait()

        def w_start(c, b):
            pltpu.async_copy(w_t.at[pl.ds(2 * (gb0 + c * CHM), 2 * CHM)],
                             wbufs[b], semw[b])

        def w_wait(c, b):
            pltpu.make_async_copy(
                w_t.at[pl.ds(2 * (gb0 + c * CHM), 2 * CHM)],
                wbufs[b], semw[b]).wait()

        def gather_start(b):
            pltpu.async_copy(h_t.at[sbufs[b]], rows[b], semg[b])

        def gather_wait(b):
            pltpu.make_async_copy(h_t.at[sbufs[b]], rows[b], semg[b]).wait()

        def scat_start(b):
            pltpu.async_copy(rows[b], acc_sp.at[dbufs[b]], sems[b], add=True)

        def scat_wait(b):
            pltpu.make_async_copy(rows[b], acc_sp.at[dbufs[b]],
                                  sems[b]).wait()

        def compute(b):
            rb = rows[b]
            wb = wbufs[b]

            def grp(g, _):
                wv = wb[pl.ds(g * 16, 16)]
                for m in range(8):
                    j = g * 8 + m
                    wq0 = jnp.full((16,), wv[2 * m], f32)
                    wq1 = jnp.full((16,), wv[2 * m + 1], f32)
                    for k in range(4):
                        rb[j, pl.ds(k * 16, 16)] = (
                            rb[j, pl.ds(k * 16, 16)] * wq0)
                    for k in range(4, 8):
                        rb[j, pl.ds(k * 16, 16)] = (
                            rb[j, pl.ds(k * 16, 16)] * wq1)
                return 0

            lax.fori_loop(0, CHM // 8, grp, 0)

        sidx_start(0, 0)
        w_start(0, 0)
        didx_start(0, 0)
        sidx_wait(0, 0)
        gather_start(0)
        sidx_start(1, 1)

        def trip(t, _):
            for u in range(3):
                c = 3 * t + u
                b = u
                bn = (u + 1) % 3
                bp = (u + 2) % 3

                def head():
                    scat_wait(bn)

                if u < 2:
                    pl.when(t > 0)(head)
                else:
                    head()

                def fill():
                    didx_start(c + 1, bn)
                    w_start(c + 1, bn)
                    sidx_wait(c + 1, bn)
                    gather_start(bn)

                if u < 2:
                    fill()
                else:
                    pl.when(t < nt - 1)(fill)

                def fill2():
                    sidx_start(c + 2, bp)

                if u == 0:
                    fill2()
                else:
                    pl.when(t < nt - 1)(fill2)

                gather_wait(b)
                w_wait(c, b)
                compute(b)
                didx_wait(c, b)
                scat_start(b)
            return 0

        lax.fori_loop(0, nt, trip, 0)
        scat_wait((nch - 2) % 3)
        scat_wait((nch - 1) % 3)
        plsc.subcore_barrier()

        pltpu.sync_copy(b_t, bb)

        def ep(i, _):
            rb = rbase + i * 64
            pltpu.sync_copy(acc_sp.at[pl.ds(rb, 64)], accb)
            pltpu.sync_copy(den_t.at[pl.ds(2 * rb, 128)], denb)

            def rgrp(g, _):
                dv = denb[pl.ds(g * 16, 16)]
                invv = 1.0 / (dv + 1e-16)
                for m in range(8):
                    r = g * 8 + m
                    inv0 = jnp.full((16,), invv[2 * m], f32)
                    inv1 = jnp.full((16,), invv[2 * m + 1], f32)
                    for k in range(8):
                        inv = inv0 if k < 4 else inv1
                        v = (accb[r, pl.ds(k * 16, 16)] * inv
                             + bb[pl.ds(k * 16, 16)])
                        v = (jnp.maximum(v, 0.0)
                             + jnp.minimum(
                                 jnp.exp(jnp.minimum(v, 0.0)) - 1.0, 0.0))
                        accb[r, pl.ds(k * 16, 16)] = v
                return 0

            lax.fori_loop(0, 8, rgrp, 0)
            pltpu.sync_copy(accb, out.at[slot, pl.ds(rb, 64)])
            return 0

        lax.fori_loop(0, rows_per_tile // 64, ep, 0)

    @pl.when(cid == 0)
    def _():
        run(wA, hA, denA, bA, 0)

    @pl.when(cid == 1)
    def _():
        run(wB, hB, denB, bB, 1)


def _msg(srcP, dstP, wA, wB, hA, hB, denA, denB, bA, bB, zz):
    k = pl.kernel(
        _msg_body,
        out_type=jax.ShapeDtypeStruct((2, NP, 128), f32),
        mesh=_MESH,
        scratch_types=(
            [pltpu.VMEM((CHM,), i32) for _ in range(6)]
            + [pltpu.VMEM((2 * CHM,), f32) for _ in range(3)]
            + [pltpu.VMEM((CHM, 128), f32) for _ in range(3)]
            + [pltpu.VMEM((64, 128), f32),
               pltpu.VMEM((128,), f32),
               pltpu.VMEM((128,), f32),
               pltpu.VMEM_SHARED((NP, 128), f32)]
            + [pltpu.SemaphoreType.DMA for _ in range(15)]
        ),
    )
    return k(srcP, dstP, wA, wB, hA, hB, denA, denB, bA, bB, zz)


def _msg1_body(src_h, dst_h, w1, h3, zz,
               out,
               sbuf, dbuf, wbuf, rowbuf, acc_sp, sem):
    cid = lax.axis_index("c")
    sid = lax.axis_index("s")
    ept = EP // (2 * NS)
    rows_per_tile = NP // NS
    rbase = sid * rows_per_tile
    rslice = pl.ds(rbase, rows_per_tile)

    pltpu.sync_copy(zz.at[rslice], acc_sp.at[rslice])
    plsc.subcore_barrier()

    tbase = (cid * NS + sid) * ept

    def chunk(i, _):
        gbase = tbase + i * CH
        pltpu.sync_copy(src_h.at[pl.ds(gbase, CH)], sbuf)
        pltpu.sync_copy(dst_h.at[pl.ds(gbase, CH)], dbuf)
        pltpu.sync_copy(w1.at[pl.ds(gbase, CH)], wbuf)
        pltpu.async_copy(h3.at[sbuf], rowbuf, sem).wait()

        def grp(g, _):
            wv = wbuf[pl.ds(g * 16, 16)]
            for m in range(16):
                j = g * 16 + m
                w0 = jnp.full((16,), wv[m], f32)
                for k in range(4):
                    rowbuf[j, pl.ds(k * 16, 16)] = (
                        rowbuf[j, pl.ds(k * 16, 16)] * w0)
            return 0

        lax.fori_loop(0, CH // 16, grp, 0)
        pltpu.sync_copy(rowbuf, acc_sp.at[dbuf], add=True)
        return 0

    lax.fori_loop(0, ept // CH, chunk, 0)
    plsc.subcore_barrier()
    pltpu.sync_copy(acc_sp.at[rslice], out.at[cid, rslice])


def _msg1(srcP, dstP, w1, h3, zz):
    k = pl.kernel(
        _msg1_body,
        out_type=jax.ShapeDtypeStruct((2, NP, 128), f32),
        mesh=_MESH,
        scratch_types=[
            pltpu.VMEM((CH,), i32),
            pltpu.VMEM((CH,), i32),
            pltpu.VMEM((CH,), f32),
            pltpu.VMEM((CH, 128), f32),
            pltpu.VMEM_SHARED((NP, 128), f32),
            pltpu.SemaphoreType.DMA,
        ],
    )
    return k(srcP, dstP, w1, h3, zz)


def _final_body(acc_ref, den_ref, b3_ref, bb_ref, linw_ref, linb_ref,
                out_ref, ssum, scnt):
    i = pl.program_id(0)

    @pl.when(i == 0)
    def _():
        ssum[...] = jnp.zeros((G, HID), f32)
        scnt[...] = jnp.zeros((G, HID), f32)

    den = den_ref[0, :, 0:1] + den_ref[1, :, 0:1]
    out3 = (acc_ref[0] + acc_ref[1]) / (den + 1e-16) + b3_ref[...]
    oneh = (bb_ref[...] == lax.broadcasted_iota(i32, (RB, G), 1)).astype(f32)
    ssum[...] += lax.dot_general(oneh, out3, (((0,), (0,)), ((), ())),
                                 preferred_element_type=f32)
    scnt[...] += lax.dot_general(oneh, jnp.ones((RB, HID), f32),
                                 (((0,), (0,)), ((), ())),
                                 preferred_element_type=f32)

    @pl.when(i == NP // RB - 1)
    def _():
        pooled = ssum[...] / jnp.maximum(scnt[...], 1.0)
        out_ref[...] = (jnp.dot(pooled, linw_ref[...],
                                preferred_element_type=f32)
                        + linb_ref[...])


def _final(acc3p, den3, b3r, batchB, linWp, linb2):
    return pl.pallas_call(
        _final_body,
        grid=(NP // RB,),
        in_specs=[
            pl.BlockSpec((2, RB, HID), lambda i: (0, i, 0)),
            pl.BlockSpec((2, RB, 8), lambda i: (0, i, 0)),
            pl.BlockSpec((1, HID), lambda i: (0, 0)),
            pl.BlockSpec((RB, G), lambda i: (i, 0)),
            pl.BlockSpec((HID, 128), lambda i: (0, 0)),
            pl.BlockSpec((1, 128), lambda i: (0, 0)),
        ],
        out_specs=pl.BlockSpec((G, 128), lambda i: (0, 0)),
        out_shape=jax.ShapeDtypeStruct((G, 128), f32),
        scratch_shapes=[pltpu.VMEM((G, HID), f32), pltpu.VMEM((G, HID), f32)],
    )(acc3p, den3, b3r, batchB, linWp, linb2)


def _perm_rows(W):
    return jnp.concatenate(
        [W[0:128], W[256:384], W[128:256], W[384:512]], axis=0)


def kernel(x, edge_index, batch, W1, a_s1, a_d1, b1, W2, a_s2, a_d2, b2,
           W3, a_s3, a_d3, b3, linW, linb):
    loops = jnp.arange(N, dtype=edge_index.dtype)
    srcP = jnp.concatenate(
        [edge_index[0], loops,
         jnp.zeros((EP - E2,), edge_index.dtype)])
    dstP = jnp.concatenate(
        [edge_index[1], loops,
         jnp.zeros((EP - E2,), edge_index.dtype)])
    xP = jnp.pad(x, ((0, NP - N), (0, 0)))
    zz128 = jnp.zeros((NP, 128), f32)
    zzN = jnp.zeros((NP,), f32)

    def gat_layer(xb, Wp, a_s, a_d, b):
        h4, alph = _dense(xb, Wp, a_s.reshape(1, 512), a_d.reshape(1, 512),
                          512, HEADS)
        w8, den8 = _edge_weights(srcP, dstP,
                                 alph[:, 0:4].reshape(-1),
                                 alph[:, 8:12].reshape(-1),
                                 alph[:, 4:8].reshape(-1),
                                 alph[:, 12:16].reshape(-1), zzN)
        outs = []
        for q in range(2):
            outs.append(_msg(
                srcP, dstP,
                jnp.stack([w8[0, 2 * q], w8[0, 2 * q + 1]], 1).reshape(-1),
                jnp.stack([w8[1, 2 * q], w8[1, 2 * q + 1]], 1).reshape(-1),
                h4[q], h4[2 + q],
                jnp.stack([den8[0, 2 * q], den8[0, 2 * q + 1]], 1).reshape(-1),
                jnp.stack([den8[1, 2 * q], den8[1, 2 * q + 1]], 1).reshape(-1),
                b[128 * q:128 * (q + 1)], b[128 * (2 + q):128 * (3 + q)],
                zz128))
        return jnp.concatenate(outs, axis=0)

    xb1 = xP[None]
    o1 = gat_layer(xb1, W1, a_s1, a_d1, b1)
    o2 = gat_layer(o1, _perm_rows(W2), a_s2, a_d2, b2)

    h3p, alph3 = _dense(o2, jnp.pad(_perm_rows(W3), ((0, 0), (0, 0))),
                        a_s3.reshape(1, HID), a_d3.reshape(1, HID),
                        HID, 1)
    w1d, den3 = _edge_weights1(srcP, dstP, alph3[:, 0], alph3[:, 8], zzN)
    acc3p = _msg1(srcP, dstP, w1d, h3p, zz128)[:, :, :HID]
    den3b = jnp.broadcast_to(den3[:, :, None], (2, NP, 8)) + 0.0

    batchP = jnp.pad(batch, (0, NP - N), constant_values=G).astype(i32)
    batchB = jnp.broadcast_to(batchP[:, None], (NP, G))
    linWp = jnp.pad(linW, ((0, 0), (0, 128 - linW.shape[1])))
    linb2 = jnp.pad(linb.reshape(1, 1), ((0, 0), (0, 127)))
    res = _final(acc3p, den3b, b3.reshape(1, HID) + 0.0, batchB, linWp, linb2)
    return res[:, :1]


# speedup vs baseline: 27.1026x; 1.1888x over previous
import functools

import jax
import jax.numpy as jnp
from jax import lax
from jax.experimental import pallas as pl
from jax.experimental.pallas import tpu as pltpu
from jax.experimental.pallas import tpu_sc as plsc

N = 10000
E = 320000
D = 128
HID = 64
HEADS = 8
G = 128

E2 = E + N
NP = 10240
EP = 331776
CH = 128
CHM = 96
RB = 1024
NS = 16
LEAK = 0.2

f32 = jnp.float32
i32 = jnp.int32

_MESH = plsc.VectorSubcoreMesh(core_axis_name="c", subcore_axis_name="s")


def _dense_body(Bin, F, H, xb_ref, w_ref, asf_ref, adf_ref, h_ref, alph_ref):
    r = jnp.dot(xb_ref[0], w_ref[0:128, :], preferred_element_type=f32)
    for j in range(1, Bin):
        r += jnp.dot(xb_ref[j], w_ref[128 * j:128 * (j + 1), :],
                     preferred_element_type=f32)
    if F == 512:
        for j in range(4):
            h_ref[j] = r[:, 128 * j:128 * (j + 1)]
    else:
        h_ref[...] = jnp.concatenate(
            [r, jnp.zeros((RB, 128 - F), f32)], axis=1)
    rs = r * asf_ref[...]
    rd = r * adf_ref[...]
    cols = []
    for h in range(H):
        cols.append(jnp.sum(rs[:, HID * h:HID * (h + 1)], axis=1,
                            keepdims=True))
    if H < 8:
        cols.append(jnp.zeros((RB, 8 - H), f32))
    for h in range(H):
        cols.append(jnp.sum(rd[:, HID * h:HID * (h + 1)], axis=1,
                            keepdims=True))
    if H < 8:
        cols.append(jnp.zeros((RB, 8 - H), f32))
    cols.append(jnp.zeros((RB, 112), f32))
    alph_ref[...] = jnp.concatenate(cols, axis=1)


def _dense(xb, Wp, asf, adf, F, H):
    Bin = xb.shape[0]
    if F == 512:
        h_shape = jax.ShapeDtypeStruct((4, NP, 128), f32)
        h_spec = pl.BlockSpec((4, RB, 128), lambda i: (0, i, 0))
    else:
        h_shape = jax.ShapeDtypeStruct((NP, 128), f32)
        h_spec = pl.BlockSpec((RB, 128), lambda i: (i, 0))
    return pl.pallas_call(
        functools.partial(_dense_body, Bin, F, H),
        grid=(NP // RB,),
        in_specs=[
            pl.BlockSpec((Bin, RB, 128), lambda i: (0, i, 0)),
            pl.BlockSpec((Bin * 128, F), lambda i: (0, 0)),
            pl.BlockSpec((1, F), lambda i: (0, 0)),
            pl.BlockSpec((1, F), lambda i: (0, 0)),
        ],
        out_specs=[h_spec, pl.BlockSpec((RB, 128), lambda i: (i, 0))],
        out_shape=[h_shape, jax.ShapeDtypeStruct((NP, 128), f32)],
    )(xb, Wp, asf, adf)


def _edge_weights_body(src_h, dst_h, asA, adA, asB, adB, zzN,
                       w_out, den_out,
                       tas, tad, s0, s1, dd0, dd1, e0, e1,
                       w00, w01, w02, w03, w10, w11, w12, w13,
                       d0, d1, d2, d3,
                       ki0, ki1, kj0, kj1, kd0, kd1,
                       kw00, kw01, kw02, kw03, kw10, kw11, kw12, kw13,
                       kc00, kc01, kc02, kc03, kc10, kc11, kc12, kc13):
    cid = lax.axis_index("c")
    sid = lax.axis_index("s")
    ept = EP // NS
    rows_per_tile = NP // NS
    sbufs, dbufs, dscat = (s0, s1), (dd0, dd1), (e0, e1)
    wbufs = ((w00, w01, w02, w03), (w10, w11, w12, w13))
    dens = (d0, d1, d2, d3)
    semsi, semdi, semds = (ki0, ki1), (kj0, kj1), (kd0, kd1)
    semw = ((kw00, kw01, kw02, kw03), (kw10, kw11, kw12, kw13))
    semc = ((kc00, kc01, kc02, kc03), (kc10, kc11, kc12, kc13))

    @pl.when(cid == 0)
    def _():
        pltpu.sync_copy(asA, tas)
        pltpu.sync_copy(adA, tad)

    @pl.when(cid == 1)
    def _():
        pltpu.sync_copy(asB, tas)
        pltpu.sync_copy(adB, tad)

    rslice = pl.ds(sid * rows_per_tile, rows_per_tile)
    for hh in range(4):
        pltpu.sync_copy(zzN.at[rslice], dens[hh].at[rslice])
    plsc.subcore_barrier()

    nch = ept // CH
    npair = nch // 2

    def sidx_start(c, b):
        pltpu.async_copy(src_h.at[pl.ds(sid * ept + c * CH, CH)],
                         sbufs[b], semsi[b])

    def sidx_wait(c, b):
        pltpu.make_async_copy(src_h.at[pl.ds(sid * ept + c * CH, CH)],
                              sbufs[b], semsi[b]).wait()

    def didx_start(c, b):
        pltpu.async_copy(dst_h.at[pl.ds(sid * ept + c * CH, CH)],
                         dbufs[b], semdi[b])

    def didx_wait(c, b):
        pltpu.make_async_copy(dst_h.at[pl.ds(sid * ept + c * CH, CH)],
                              dbufs[b], semdi[b]).wait()

    def dscat_start(c, b):
        pltpu.async_copy(dst_h.at[pl.ds(sid * ept + c * CH, CH)],
                         dscat[b], semds[b])

    def dscat_wait(c, b):
        pltpu.make_async_copy(dst_h.at[pl.ds(sid * ept + c * CH, CH)],
                              dscat[b], semds[b]).wait()

    def wout_start(c, b, hh):
        pltpu.async_copy(wbufs[b][hh],
                         w_out.at[cid, hh, pl.ds(sid * ept + c * CH, CH)],
                         semw[b][hh])

    def wout_wait(c, b, hh):
        pltpu.make_async_copy(
            wbufs[b][hh],
            w_out.at[cid, hh, pl.ds(sid * ept + c * CH, CH)],
            semw[b][hh]).wait()

    def scat_start(b, hh):
        pltpu.async_copy(wbufs[b][hh], dens[hh].at[dscat[b]],
                         semc[b][hh], add=True)

    def scat_wait(b, hh):
        pltpu.make_async_copy(wbufs[b][hh], dens[hh].at[dscat[b]],
                              semc[b][hh]).wait()

    def compute(c, b):
        gbase = sid * ept + c * CH

        def vec(j, _):
            s16 = sbufs[b][pl.ds(j * 16, 16)] * 4
            d16 = dbufs[b][pl.ds(j * 16, 16)] * 4
            ids = gbase + j * 16 + lax.iota(i32, 16)
            mf = jnp.clip(E2 - ids, 0, 1).astype(f32)
            for hh in range(4):
                e = (plsc.load_gather(tas, [s16 + hh])
                     + plsc.load_gather(tad, [d16 + hh]))
                e = jnp.maximum(e, LEAK * e)
                wbufs[b][hh][pl.ds(j * 16, 16)] = jnp.exp(e) * mf
            return 0

        lax.fori_loop(0, CH // 16, vec, 0)

    sidx_start(0, 0)
    didx_start(0, 0)
    sidx_start(1, 1)
    didx_start(1, 1)

    def pair(t, _):
        for u in range(2):
            c = 2 * t + u
            b = u

            def drain():
                for hh in range(4):
                    wout_wait(c - 2, b, hh)
                    scat_wait(b, hh)

            pl.when(t > 0)(drain)

            dscat_start(c, b)
            sidx_wait(c, b)
            didx_wait(c, b)
            compute(c, b)

            def fill():
                sidx_start(c + 2, b)
                didx_start(c + 2, b)

            pl.when(t < npair - 1)(fill)

            dscat_wait(c, b)
            for hh in range(4):
                wout_start(c, b, hh)
                scat_start(b, hh)
        return 0

    lax.fori_loop(0, npair, pair, 0)
    for b in range(2):
        for hh in range(4):
            wout_wait(nch - 2 + b, b, hh)
            scat_wait(b, hh)
    plsc.subcore_barrier()
    for hh in range(4):
        pltpu.sync_copy(dens[hh].at[rslice], den_out.at[cid, hh, rslice])


def _edge_weights(srcP, dstP, asA, adA, asB, adB, zzN):
    k = pl.kernel(
        _edge_weights_body,
        out_type=[jax.ShapeDtypeStruct((2, 4, EP), f32),
                  jax.ShapeDtypeStruct((2, 4, NP), f32)],
        mesh=_MESH,
        compiler_params=pltpu.CompilerParams(needs_layout_passes=False),
        scratch_types=(
            [pltpu.VMEM((NP * 4,), f32) for _ in range(2)]
            + [pltpu.VMEM((CH,), i32) for _ in range(6)]
            + [pltpu.VMEM((CH,), f32) for _ in range(8)]
            + [pltpu.VMEM_SHARED((NP,), f32) for _ in range(4)]
            + [pltpu.SemaphoreType.DMA for _ in range(22)]
        ),
    )
    return k(srcP, dstP, asA, adA, asB, adB, zzN)


def _edge_weights1_body(src_h, dst_h, as1, ad1, zzN,
                        w_out, den_out,
                        tas, tad, sbuf, dbuf, wflat, den_sp, sem):
    cid = lax.axis_index("c")
    sid = lax.axis_index("s")
    ept = EP // (2 * NS)
    rows_per_tile = NP // NS

    pltpu.sync_copy(as1, tas)
    pltpu.sync_copy(ad1, tad)
    rslice = pl.ds(sid * rows_per_tile, rows_per_tile)
    pltpu.sync_copy(zzN.at[rslice], den_sp.at[rslice])
    plsc.subcore_barrier()

    tbase = (cid * NS + sid) * ept

    def chunk(i, _):
        gbase = tbase + i * CH
        pltpu.sync_copy(src_h.at[pl.ds(gbase, CH)], sbuf)
        pltpu.sync_copy(dst_h.at[pl.ds(gbase, CH)], dbuf)

        def vec(j, _):
            s16 = sbuf[pl.ds(j * 16, 16)]
            d16 = dbuf[pl.ds(j * 16, 16)]
            ids = gbase + j * 16 + lax.iota(i32, 16)
            mf = jnp.clip(E2 - ids, 0, 1).astype(f32)
            e = plsc.load_gather(tas, [s16]) + plsc.load_gather(tad, [d16])
            e = jnp.maximum(e, LEAK * e)
            wflat[pl.ds(j * 16, 16)] = jnp.exp(e) * mf
            return 0

        lax.fori_loop(0, CH // 16, vec, 0)
        pltpu.sync_copy(wflat, w_out.at[pl.ds(gbase, CH)])
        pltpu.sync_copy(wflat, den_sp.at[dbuf], add=True)
        return 0

    lax.fori_loop(0, ept // CH, chunk, 0)
    plsc.subcore_barrier()
    pltpu.sync_copy(den_sp.at[rslice], den_out.at[cid, rslice])


def _edge_weights1(srcP, dstP, as1, ad1, zzN):
    k = pl.kernel(
        _edge_weights1_body,
        out_type=[jax.ShapeDtypeStruct((EP,), f32),
                  jax.ShapeDtypeStruct((2, NP), f32)],
        mesh=_MESH,
        compiler_params=pltpu.CompilerParams(needs_layout_passes=False),
        scratch_types=[
            pltpu.VMEM((NP,), f32),
            pltpu.VMEM((NP,), f32),
            pltpu.VMEM((CH,), i32),
            pltpu.VMEM((CH,), i32),
            pltpu.VMEM((CH,), f32),
            pltpu.VMEM_SHARED((NP,), f32),
            pltpu.SemaphoreType.DMA,
        ],
    )
    return k(srcP, dstP, as1, ad1, zzN)


def _msg_body(src_h, dst_h, wA, wB, hA, hB, denA, denB,
              bA, bB, zz,
              out,
              s0, s1, s2, d0, d1, d2, w0b, w1b, w2b, r0, r1, r2,
              accb, denb, bb, acc_sp,
              sg0, sg1, sg2, ss0, ss1, ss2,
              sw0, sw1, sw2, si0, si1, si2, sj0, sj1, sj2):
    cid = lax.axis_index("c")
    sid = lax.axis_index("s")
    ept = EP // NS
    rows_per_tile = NP // NS
    rbase = sid * rows_per_tile
    sbufs, dbufs = (s0, s1, s2), (d0, d1, d2)
    wbufs, rows = (w0b, w1b, w2b), (r0, r1, r2)
    semg, sems, semw = (sg0, sg1, sg2), (ss0, ss1, ss2), (sw0, sw1, sw2)
    semsi, semdi = (si0, si1, si2), (sj0, sj1, sj2)

    def run(w_t, h_t, den_t, b_t, slot):
        pltpu.sync_copy(zz.at[pl.ds(rbase, rows_per_tile)],
                        acc_sp.at[pl.ds(rbase, rows_per_tile)])
        plsc.subcore_barrier()

        gb0 = sid * ept
        nch = ept // CHM
        nt = nch // 3

        def sidx_start(c, b):
            pltpu.async_copy(src_h.at[pl.ds(gb0 + c * CHM, CHM)],
                             sbufs[b], semsi[b])

        def sidx_wait(c, b):
            pltpu.make_async_copy(src_h.at[pl.ds(gb0 + c * CHM, CHM)],
                                  sbufs[b], semsi[b]).wait()

        def didx_start(c, b):
            pltpu.async_copy(dst_h.at[pl.ds(gb0 + c * CHM, CHM)],
                             dbufs[b], semdi[b])

        def didx_wait(c, b):
            pltpu.make_async_copy(dst_h.at[pl.ds(gb0 + c * CHM, CHM)],
                                  dbufs[b], semdi[b]).wait()

        def w_start(c, b):
            pltpu.async_copy(w_t.at[pl.ds(2 * (gb0 + c * CHM), 2 * CHM)],
                             wbufs[b], semw[b])

        def w_wait(c, b):
            pltpu.make_async_copy(
                w_t.at[pl.ds(2 * (gb0 + c * CHM), 2 * CHM)],
                wbufs[b], semw[b]).wait()

        def gather_start(b):
            pltpu.async_copy(h_t.at[sbufs[b]], rows[b], semg[b])

        def gather_wait(b):
            pltpu.make_async_copy(h_t.at[sbufs[b]], rows[b], semg[b]).wait()

        def scat_start(b):
            pltpu.async_copy(rows[b], acc_sp.at[dbufs[b]], sems[b], add=True)

        def scat_wait(b):
            pltpu.make_async_copy(rows[b], acc_sp.at[dbufs[b]],
                                  sems[b]).wait()

        def compute(b):
            rb = rows[b]
            wb = wbufs[b]

            def grp(g, _):
                wv = wb[pl.ds(g * 16, 16)]
                for m in range(8):
                    j = g * 8 + m
                    wq0 = jnp.full((16,), wv[2 * m], f32)
                    wq1 = jnp.full((16,), wv[2 * m + 1], f32)
                    for k in range(4):
                        rb[j, pl.ds(k * 16, 16)] = (
                            rb[j, pl.ds(k * 16, 16)] * wq0)
                    for k in range(4, 8):
                        rb[j, pl.ds(k * 16, 16)] = (
                            rb[j, pl.ds(k * 16, 16)] * wq1)
                return 0

            lax.fori_loop(0, CHM // 8, grp, 0)

        sidx_start(0, 0)
        w_start(0, 0)
        didx_start(0, 0)
        sidx_wait(0, 0)
        gather_start(0)
        sidx_start(1, 1)

        def trip(t, _):
            for u in range(3):
                c = 3 * t + u
                b = u
                bn = (u + 1) % 3
                bp = (u + 2) % 3

                def head():
                    scat_wait(bn)

                if u < 2:
                    pl.when(t > 0)(head)
                else:
                    head()

                def fill():
                    didx_start(c + 1, bn)
                    w_start(c + 1, bn)
                    sidx_wait(c + 1, bn)
                    gather_start(bn)

                if u < 2:
                    fill()
                else:
                    pl.when(t < nt - 1)(fill)

                def fill2():
                    sidx_start(c + 2, bp)

                if u == 0:
                    fill2()
                else:
                    pl.when(t < nt - 1)(fill2)

                gather_wait(b)
                w_wait(c, b)
                compute(b)
                didx_wait(c, b)
                scat_start(b)
            return 0

        lax.fori_loop(0, nt, trip, 0)
        scat_wait((nch - 2) % 3)
        scat_wait((nch - 1) % 3)
        plsc.subcore_barrier()

        pltpu.sync_copy(b_t, bb)

        def ep(i, _):
            rb = rbase + i * 64
            pltpu.sync_copy(acc_sp.at[pl.ds(rb, 64)], accb)
            pltpu.sync_copy(den_t.at[pl.ds(2 * rb, 128)], denb)

            def rgrp(g, _):
                dv = denb[pl.ds(g * 16, 16)]
                invv = 1.0 / (dv + 1e-16)
                for m in range(8):
                    r = g * 8 + m
                    inv0 = jnp.full((16,), invv[2 * m], f32)
                    inv1 = jnp.full((16,), invv[2 * m + 1], f32)
                    for k in range(8):
                        inv = inv0 if k < 4 else inv1
                        v = (accb[r, pl.ds(k * 16, 16)] * inv
                             + bb[pl.ds(k * 16, 16)])
                        v = (jnp.maximum(v, 0.0)
                             + jnp.minimum(
                                 jnp.exp(jnp.minimum(v, 0.0)) - 1.0, 0.0))
                        accb[r, pl.ds(k * 16, 16)] = v
                return 0

            lax.fori_loop(0, 8, rgrp, 0)
            pltpu.sync_copy(accb, out.at[slot, pl.ds(rb, 64)])
            return 0

        lax.fori_loop(0, rows_per_tile // 64, ep, 0)

    @pl.when(cid == 0)
    def _():
        run(wA, hA, denA, bA, 0)

    @pl.when(cid == 1)
    def _():
        run(wB, hB, denB, bB, 1)


def _msg(srcP, dstP, wA, wB, hA, hB, denA, denB, bA, bB, zz):
    k = pl.kernel(
        _msg_body,
        out_type=jax.ShapeDtypeStruct((2, NP, 128), f32),
        mesh=_MESH,
        scratch_types=(
            [pltpu.VMEM((CHM,), i32) for _ in range(6)]
            + [pltpu.VMEM((2 * CHM,), f32) for _ in range(3)]
            + [pltpu.VMEM((CHM, 128), f32) for _ in range(3)]
            + [pltpu.VMEM((64, 128), f32),
               pltpu.VMEM((128,), f32),
               pltpu.VMEM((128,), f32),
               pltpu.VMEM_SHARED((NP, 128), f32)]
            + [pltpu.SemaphoreType.DMA for _ in range(15)]
        ),
    )
    return k(srcP, dstP, wA, wB, hA, hB, denA, denB, bA, bB, zz)


def _msg1_body(src_h, dst_h, w1, h3, zz,
               out,
               s0, s1, s2, d0, d1, d2, w0b, w1b, w2b, r0, r1, r2,
               acc_sp,
               sg0, sg1, sg2, ss0, ss1, ss2,
               sw0, sw1, sw2, si0, si1, si2, sj0, sj1, sj2):
    cid = lax.axis_index("c")
    sid = lax.axis_index("s")
    ept = EP // (2 * NS)
    rows_per_tile = NP // NS
    rbase = sid * rows_per_tile
    rslice = pl.ds(rbase, rows_per_tile)
    sbufs, dbufs = (s0, s1, s2), (d0, d1, d2)
    wbufs, rows = (w0b, w1b, w2b), (r0, r1, r2)
    semg, sems, semw = (sg0, sg1, sg2), (ss0, ss1, ss2), (sw0, sw1, sw2)
    semsi, semdi = (si0, si1, si2), (sj0, sj1, sj2)

    pltpu.sync_copy(zz.at[rslice], acc_sp.at[rslice])
    plsc.subcore_barrier()

    gb0 = (cid * NS + sid) * ept
    nch = ept // CHM
    nt = nch // 3

    def sidx_start(c, b):
        pltpu.async_copy(src_h.at[pl.ds(gb0 + c * CHM, CHM)],
                         sbufs[b], semsi[b])

    def sidx_wait(c, b):
        pltpu.make_async_copy(src_h.at[pl.ds(gb0 + c * CHM, CHM)],
                              sbufs[b], semsi[b]).wait()

    def didx_start(c, b):
        pltpu.async_copy(dst_h.at[pl.ds(gb0 + c * CHM, CHM)],
                         dbufs[b], semdi[b])

    def didx_wait(c, b):
        pltpu.make_async_copy(dst_h.at[pl.ds(gb0 + c * CHM, CHM)],
                              dbufs[b], semdi[b]).wait()

    def w_start(c, b):
        pltpu.async_copy(w1.at[pl.ds(gb0 + c * CHM, CHM)],
                         wbufs[b], semw[b])

    def w_wait(c, b):
        pltpu.make_async_copy(w1.at[pl.ds(gb0 + c * CHM, CHM)],
                              wbufs[b], semw[b]).wait()

    def gather_start(b):
        pltpu.async_copy(h3.at[sbufs[b]], rows[b], semg[b])

    def gather_wait(b):
        pltpu.make_async_copy(h3.at[sbufs[b]], rows[b], semg[b]).wait()

    def scat_start(b):
        pltpu.async_copy(rows[b], acc_sp.at[dbufs[b]], sems[b], add=True)

    def scat_wait(b):
        pltpu.make_async_copy(rows[b], acc_sp.at[dbufs[b]], sems[b]).wait()

    def compute(b):
        rb = rows[b]
        wb = wbufs[b]

        def grp(g, _):
            wv = wb[pl.ds(g * 16, 16)]
            for m in range(16):
                j = g * 16 + m
                w0 = jnp.full((16,), wv[m], f32)
                for k in range(4):
                    rb[j, pl.ds(k * 16, 16)] = (
                        rb[j, pl.ds(k * 16, 16)] * w0)
            return 0

        lax.fori_loop(0, CHM // 16, grp, 0)

    sidx_start(0, 0)
    w_start(0, 0)
    didx_start(0, 0)
    sidx_wait(0, 0)
    gather_start(0)
    sidx_start(1, 1)

    def trip(t, _):
        for u in range(3):
            c = 3 * t + u
            b = u
            bn = (u + 1) % 3
            bp = (u + 2) % 3

            def head():
                scat_wait(bn)

            if u < 2:
                pl.when(t > 0)(head)
            else:
                head()

            def fill():
                didx_start(c + 1, bn)
                w_start(c + 1, bn)
                sidx_wait(c + 1, bn)
                gather_start(bn)

            if u < 2:
                fill()
            else:
                pl.when(t < nt - 1)(fill)

            def fill2():
                sidx_start(c + 2, bp)

            if u == 0:
                fill2()
            else:
                pl.when(t < nt - 1)(fill2)

            gather_wait(b)
            w_wait(c, b)
            compute(b)
            didx_wait(c, b)
            scat_start(b)
        return 0

    lax.fori_loop(0, nt, trip, 0)
    scat_wait((nch - 2) % 3)
    scat_wait((nch - 1) % 3)
    plsc.subcore_barrier()
    pltpu.sync_copy(acc_sp.at[rslice], out.at[cid, rslice])


def _msg1(srcP, dstP, w1, h3, zz):
    k = pl.kernel(
        _msg1_body,
        out_type=jax.ShapeDtypeStruct((2, NP, 128), f32),
        mesh=_MESH,
        scratch_types=(
            [pltpu.VMEM((CHM,), i32) for _ in range(6)]
            + [pltpu.VMEM((CHM,), f32) for _ in range(3)]
            + [pltpu.VMEM((CHM, 128), f32) for _ in range(3)]
            + [pltpu.VMEM_SHARED((NP, 128), f32)]
            + [pltpu.SemaphoreType.DMA for _ in range(15)]
        ),
    )
    return k(srcP, dstP, w1, h3, zz)


def _final_body(acc_ref, den_ref, b3_ref, bb_ref, linw_ref, linb_ref,
                out_ref, ssum, scnt):
    i = pl.program_id(0)

    @pl.when(i == 0)
    def _():
        ssum[...] = jnp.zeros((G, HID), f32)
        scnt[...] = jnp.zeros((G, HID), f32)

    den = den_ref[0, :, 0:1] + den_ref[1, :, 0:1]
    out3 = (acc_ref[0] + acc_ref[1]) / (den + 1e-16) + b3_ref[...]
    oneh = (bb_ref[...] == lax.broadcasted_iota(i32, (RB, G), 1)).astype(f32)
    ssum[...] += lax.dot_general(oneh, out3, (((0,), (0,)), ((), ())),
                                 preferred_element_type=f32)
    scnt[...] += lax.dot_general(oneh, jnp.ones((RB, HID), f32),
                                 (((0,), (0,)), ((), ())),
                                 preferred_element_type=f32)

    @pl.when(i == NP // RB - 1)
    def _():
        pooled = ssum[...] / jnp.maximum(scnt[...], 1.0)
        out_ref[...] = (jnp.dot(pooled, linw_ref[...],
                                preferred_element_type=f32)
                        + linb_ref[...])


def _final(acc3p, den3, b3r, batchB, linWp, linb2):
    return pl.pallas_call(
        _final_body,
        grid=(NP // RB,),
        in_specs=[
            pl.BlockSpec((2, RB, HID), lambda i: (0, i, 0)),
            pl.BlockSpec((2, RB, 8), lambda i: (0, i, 0)),
            pl.BlockSpec((1, HID), lambda i: (0, 0)),
            pl.BlockSpec((RB, G), lambda i: (i, 0)),
            pl.BlockSpec((HID, 128), lambda i: (0, 0)),
            pl.BlockSpec((1, 128), lambda i: (0, 0)),
        ],
        out_specs=pl.BlockSpec((G, 128), lambda i: (0, 0)),
        out_shape=jax.ShapeDtypeStruct((G, 128), f32),
        scratch_shapes=[pltpu.VMEM((G, HID), f32), pltpu.VMEM((G, HID), f32)],
    )(acc3p, den3, b3r, batchB, linWp, linb2)


def _perm_rows(W):
    return jnp.concatenate(
        [W[0:128], W[256:384], W[128:256], W[384:512]], axis=0)


def kernel(x, edge_index, batch, W1, a_s1, a_d1, b1, W2, a_s2, a_d2, b2,
           W3, a_s3, a_d3, b3, linW, linb):
    loops = jnp.arange(N, dtype=edge_index.dtype)
    srcP = jnp.concatenate(
        [edge_index[0], loops,
         jnp.zeros((EP - E2,), edge_index.dtype)])
    dstP = jnp.concatenate(
        [edge_index[1], loops,
         jnp.zeros((EP - E2,), edge_index.dtype)])
    xP = jnp.pad(x, ((0, NP - N), (0, 0)))
    zz128 = jnp.zeros((NP, 128), f32)
    zzN = jnp.zeros((NP,), f32)

    def gat_layer(xb, Wp, a_s, a_d, b):
        h4, alph = _dense(xb, Wp, a_s.reshape(1, 512), a_d.reshape(1, 512),
                          512, HEADS)
        w8, den8 = _edge_weights(srcP, dstP,
                                 alph[:, 0:4].reshape(-1),
                                 alph[:, 8:12].reshape(-1),
                                 alph[:, 4:8].reshape(-1),
                                 alph[:, 12:16].reshape(-1), zzN)
        outs = []
        for q in range(2):
            outs.append(_msg(
                srcP, dstP,
                jnp.stack([w8[0, 2 * q], w8[0, 2 * q + 1]], 1).reshape(-1),
                jnp.stack([w8[1, 2 * q], w8[1, 2 * q + 1]], 1).reshape(-1),
                h4[q], h4[2 + q],
                jnp.stack([den8[0, 2 * q], den8[0, 2 * q + 1]], 1).reshape(-1),
                jnp.stack([den8[1, 2 * q], den8[1, 2 * q + 1]], 1).reshape(-1),
                b[128 * q:128 * (q + 1)], b[128 * (2 + q):128 * (3 + q)],
                zz128))
        return jnp.concatenate(outs, axis=0)

    xb1 = xP[None]
    o1 = gat_layer(xb1, W1, a_s1, a_d1, b1)
    o2 = gat_layer(o1, _perm_rows(W2), a_s2, a_d2, b2)

    h3p, alph3 = _dense(o2, jnp.pad(_perm_rows(W3), ((0, 0), (0, 0))),
                        a_s3.reshape(1, HID), a_d3.reshape(1, HID),
                        HID, 1)
    w1d, den3 = _edge_weights1(srcP, dstP, alph3[:, 0], alph3[:, 8], zzN)
    acc3p = _msg1(srcP, dstP, w1d, h3p, zz128)[:, :, :HID]
    den3b = jnp.broadcast_to(den3[:, :, None], (2, NP, 8)) + 0.0

    batchP = jnp.pad(batch, (0, NP - N), constant_values=G).astype(i32)
    batchB = jnp.broadcast_to(batchP[:, None], (NP, G))
    linWp = jnp.pad(linW, ((0, 0), (0, 128 - linW.shape[1])))
    linb2 = jnp.pad(linb.reshape(1, 1), ((0, 0), (0, 127)))
    res = _final(acc3p, den3b, b3.reshape(1, HID) + 0.0, batchB, linWp, linb2)
    return res[:, :1]
